# baseline (speedup 1.0000x reference)
import functools

import jax
import jax.numpy as jnp
from jax import lax
from jax.experimental import pallas as pl
from jax.experimental.pallas import tpu as pltpu
from jax.experimental.pallas import tpu_sc as plsc

Q = 1024
K_ORIG = 100000
NT = 32
PER = 3136
K_PAD = NT * PER
NB = 8192
LOGB = 13
QV = Q // 16

_mesh = plsc.VectorSubcoreMesh(core_axis_name="c", subcore_axis_name="s")
_params = pltpu.CompilerParams(needs_layout_passes=False)


def _build_search_body(x_hbm, keys_hbm, outd_hbm, outi_hbm,
                       keys_v, xq_v, sv_v, si_v, off_v, ctr_v, claim_v,
                       bd_v, bi_v):
    wid = lax.axis_index("s") * 2 + lax.axis_index("c")
    base = wid * PER
    iota = lax.iota(jnp.int32, 16)
    ones_i = jnp.ones((16,), jnp.int32)

    nk = jnp.where(wid == NT - 1, K_ORIG - (NT - 1) * PER, PER)
    nv = nk // 16

    @pl.when(wid == NT - 1)
    def _():
        pltpu.sync_copy(keys_hbm.at[pl.ds(base, K_ORIG - (NT - 1) * PER)],
                        keys_v.at[pl.ds(0, K_ORIG - (NT - 1) * PER)])

    @pl.when(wid != NT - 1)
    def _():
        pltpu.sync_copy(keys_hbm.at[pl.ds(base, PER)], keys_v)

    pltpu.sync_copy(x_hbm, xq_v)

    def mm_step(i, c):
        mn, mx = c
        k = keys_v[pl.ds(i * 16, 16)]
        mn = jnp.minimum(mn, k)
        mx = jnp.maximum(mx, k)
        return mn, mx

    mn0 = jnp.full((16,), jnp.inf, jnp.float32)
    mx0 = jnp.full((16,), -jnp.inf, jnp.float32)
    mn, mx = lax.fori_loop(0, nv, mm_step, (mn0, mx0))
    vmn = jnp.min(mn)
    vmx = jnp.max(mx)
    width = jnp.maximum(vmx - vmn, jnp.float32(1e-30))
    w16 = jnp.broadcast_to(width, (16,))
    ebits = (lax.bitcast_convert_type(w16, jnp.int32) >> 23) & 0xFF
    scale = lax.bitcast_convert_type((253 + LOGB - ebits) << 23, jnp.float32)
    invb = lax.bitcast_convert_type((ebits + 1 - LOGB) << 23, jnp.float32)

    def kbin(v):
        return jnp.clip((v - vmn) * scale, 0.0, float(NB - 1)).astype(jnp.int32)

    def z_step(i, _):
        z = jnp.zeros((16,), jnp.int32)
        ctr_v[pl.ds(i * 64, 16)] = z
        ctr_v[pl.ds(i * 64 + 16, 16)] = z
        ctr_v[pl.ds(i * 64 + 32, 16)] = z
        ctr_v[pl.ds(i * 64 + 48, 16)] = z
        return 0

    lax.fori_loop(0, NB // 64, z_step, 0)

    def h_step(i, _):
        k = keys_v[pl.ds(i * 16, 16)]
        plsc.addupdate_scatter(ctr_v, [kbin(k)], ones_i)
        return 0

    lax.fori_loop(0, nv, h_step, 0)

    def s_step(i, carry):
        cnt = ctr_v[pl.ds(i * 16, 16)]
        cs = plsc.cumsum(cnt)
        excl = cs - cnt + carry
        off_v[pl.ds(i * 16, 16)] = excl
        ctr_v[pl.ds(i * 16, 16)] = excl
        return carry + jnp.sum(cnt)

    lax.fori_loop(0, NB // 16, s_step, jnp.int32(0))

    def p_step(i, _):
        k = keys_v[pl.ds(i * 16, 16)]
        g = base + i * 16 + iota
        b = kbin(k)

        def cond(c):
            return jnp.any(c[0])

        def body(c):
            act = c[0]
            plsc.store_scatter(claim_v, [b], iota, mask=act)
            win = (plsc.load_gather(claim_v, [b]) == iota) & act
            pos = plsc.load_gather(ctr_v, [b])
            plsc.store_scatter(sv_v, [pos], k, mask=win)
            plsc.store_scatter(si_v, [pos], g, mask=win)
            plsc.addupdate_scatter(ctr_v, [b], ones_i, mask=win)
            return (act & jnp.logical_not(win),)

        lax.while_loop(cond, body, (jnp.ones((16,), jnp.bool_),))
        return 0

    lax.fori_loop(0, nv, p_step, 0)

    NQ = 8

    def q_step(j, _):
        qs = [xq_v[pl.ds(j * 16 * NQ + 16 * r, 16)] for r in range(NQ)]
        sqs = [(q - vmn) * scale for q in qs]
        p0s = [plsc.load_gather(off_v, [kbin(q)]) for q in qs]
        inf_v = jnp.full((16,), jnp.inf, jnp.float32)
        big_v = jnp.full((16,), 1 << 30, jnp.int32)

        def upd_best(q, act, v, gi, bd, bi):
            d = jnp.abs(q - v)
            u = act & ((d < bd) | ((d == bd) & (gi < bi)))
            return jnp.where(u, d, bd), jnp.where(u, gi, bi)

        def bi_cond(c):
            alive = None
            for (actu, _, actd, _, _, _) in c:
                a = actu | actd
                alive = a if alive is None else (alive | a)
            return jnp.any(alive)

        def one(q, sq, c):
            actu, curu, actd, curd, bd, bi = c
            vu = plsc.load_gather(sv_v, [curu])
            gu = plsc.load_gather(si_v, [curu])
            vd = plsc.load_gather(sv_v, [curd])
            gd = plsc.load_gather(si_v, [curd])
            bd, bi = upd_best(q, actu, vu, gu, bd, bi)
            bd, bi = upd_best(q, actd, vd, gd, bd, bi)
            sbd = bd * scale
            su = (vu - vmn) * scale
            sd = (vd - vmn) * scale
            actu = actu & jnp.logical_not((su - sq) - sbd > 1.05) \
                        & (curu + 1 < nk)
            actd = actd & jnp.logical_not((sq - sd) - sbd > 1.05) \
                        & (curd - 1 >= 0)
            curu = jnp.where(actu, curu + 1, curu)
            curd = jnp.where(actd, curd - 1, curd)
            return actu, curu, actd, curd, bd, bi

        def bi_body(c):
            return tuple(one(qs[r], sqs[r], c[r]) for r in range(NQ))

        def init(p0):
            return (p0 < nk, jnp.minimum(p0, nk - 1),
                    (p0 - 1) >= 0, jnp.maximum(p0 - 1, 0), inf_v, big_v)

        fin = lax.while_loop(bi_cond, bi_body,
                             tuple(init(p0s[r]) for r in range(NQ)))
        for r in range(NQ):
            bd_v[pl.ds(j * 16 * NQ + 16 * r, 16)] = fin[r][4]
            bi_v[pl.ds(j * 16 * NQ + 16 * r, 16)] = fin[r][5]
        return 0

    lax.fori_loop(0, QV // NQ, q_step, 0)

    pltpu.sync_copy(bd_v, outd_hbm.at[wid])
    pltpu.sync_copy(bi_v, outi_hbm.at[wid])


@functools.partial(
    pl.kernel, mesh=_mesh, compiler_params=_params,
    out_type=[jax.ShapeDtypeStruct((NT, Q), jnp.float32),
              jax.ShapeDtypeStruct((NT, Q), jnp.int32)],
    scratch_types=[
        pltpu.VMEM((PER,), jnp.float32),
        pltpu.VMEM((Q,), jnp.float32),
        pltpu.VMEM((PER,), jnp.float32),
        pltpu.VMEM((PER,), jnp.int32),
        pltpu.VMEM((NB,), jnp.int32),
        pltpu.VMEM((NB,), jnp.int32),
        pltpu.VMEM((NB,), jnp.int32),
        pltpu.VMEM((Q,), jnp.float32),
        pltpu.VMEM((Q,), jnp.int32),
    ],
)
def _build_search(x_hbm, keys_hbm, outd_hbm, outi_hbm,
                  keys_v, xq_v, sv_v, si_v, off_v, ctr_v, claim_v, bd_v, bi_v):
    _build_search_body(x_hbm, keys_hbm, outd_hbm, outi_hbm,
                       keys_v, xq_v, sv_v, si_v, off_v, ctr_v, claim_v,
                       bd_v, bi_v)


@functools.partial(
    pl.kernel, mesh=_mesh, compiler_params=_params,
    out_type=jax.ShapeDtypeStruct((Q,), jnp.float32),
    scratch_types=[
        pltpu.VMEM((NT, 32), jnp.float32),
        pltpu.VMEM((NT, 32), jnp.int32),
        pltpu.VMEM((32,), jnp.int32),
        pltpu.VMEM((32,), jnp.float32),
        pltpu.SemaphoreType.DMA,
    ],
)
def _merge_gather(d_hbm, i_hbm, acc_hbm, out_hbm, dbuf, ibuf, win_i, vbuf, sem):
    wid = lax.axis_index("s") * 2 + lax.axis_index("c")
    qbase = wid * 32

    copies = []
    for t in range(NT):
        copies.append(pltpu.async_copy(
            d_hbm.at[t, pl.ds(qbase, 32)], dbuf.at[t], sem))
        copies.append(pltpu.async_copy(
            i_hbm.at[t, pl.ds(qbase, 32)], ibuf.at[t], sem))
    for c in copies:
        c.wait()

    def mix(bd, bi, d, gi):
        u = (d < bd) | ((d == bd) & (gi < bi))
        return jnp.where(u, d, bd), jnp.where(u, gi, bi)

    bd0 = jnp.full((16,), jnp.inf, jnp.float32)
    bd1 = bd0
    bi0 = jnp.full((16,), 1 << 30, jnp.int32)
    bi1 = bi0
    for t in range(NT):
        bd0, bi0 = mix(bd0, bi0, dbuf[t, pl.ds(0, 16)], ibuf[t, pl.ds(0, 16)])
        bd1, bi1 = mix(bd1, bi1, dbuf[t, pl.ds(16, 16)], ibuf[t, pl.ds(16, 16)])
    win_i[pl.ds(0, 16)] = bi0
    win_i[pl.ds(16, 16)] = bi1
    pltpu.async_copy(acc_hbm.at[win_i], vbuf, sem).wait()
    pltpu.sync_copy(vbuf, out_hbm.at[pl.ds(qbase, 32)])


def kernel(x, input_tensor, accuracy_tensor):
    pd, pi = _build_search(x, input_tensor)
    return _merge_gather(pd, pi, accuracy_tensor)


# speedup vs baseline: 1.0776x; 1.0776x over previous
import functools

import jax
import jax.numpy as jnp
from jax import lax
from jax.experimental import pallas as pl
from jax.experimental.pallas import tpu as pltpu
from jax.experimental.pallas import tpu_sc as plsc

Q = 1024
K_ORIG = 100000
NT = 32
PER = 3136
K_PAD = NT * PER
NB = 8192
LOGB = 13
QV = Q // 16

_mesh = plsc.VectorSubcoreMesh(core_axis_name="c", subcore_axis_name="s")
_params = pltpu.CompilerParams(needs_layout_passes=False)


def _build_search_body(x_hbm, keys_hbm, outd_hbm, outi_hbm,
                       keys_v, xq_v, sv_v, si_v, off_v, ctr_v, claim_v,
                       bd_v, bi_v):
    wid = lax.axis_index("s") * 2 + lax.axis_index("c")
    base = wid * PER
    iota = lax.iota(jnp.int32, 16)
    ones_i = jnp.ones((16,), jnp.int32)

    nk = jnp.where(wid == NT - 1, K_ORIG - (NT - 1) * PER, PER)
    nv = nk // 16

    @pl.when(wid == NT - 1)
    def _():
        pltpu.sync_copy(keys_hbm.at[pl.ds(base, K_ORIG - (NT - 1) * PER)],
                        keys_v.at[pl.ds(0, K_ORIG - (NT - 1) * PER)])

    @pl.when(wid != NT - 1)
    def _():
        pltpu.sync_copy(keys_hbm.at[pl.ds(base, PER)], keys_v)

    pltpu.sync_copy(x_hbm, xq_v)

    def mm_step(i, c):
        mn, mx = c
        k = keys_v[pl.ds(i * 16, 16)]
        mn = jnp.minimum(mn, k)
        mx = jnp.maximum(mx, k)
        return mn, mx

    mn0 = jnp.full((16,), jnp.inf, jnp.float32)
    mx0 = jnp.full((16,), -jnp.inf, jnp.float32)
    mn, mx = lax.fori_loop(0, nv, mm_step, (mn0, mx0))
    vmn = jnp.min(mn)
    vmx = jnp.max(mx)
    width = jnp.maximum(vmx - vmn, jnp.float32(1e-30))
    w16 = jnp.broadcast_to(width, (16,))
    ebits = (lax.bitcast_convert_type(w16, jnp.int32) >> 23) & 0xFF
    scale = lax.bitcast_convert_type((253 + LOGB - ebits) << 23, jnp.float32)
    invb = lax.bitcast_convert_type((ebits + 1 - LOGB) << 23, jnp.float32)

    def kbin(v):
        return jnp.clip((v - vmn) * scale, 0.0, float(NB - 1)).astype(jnp.int32)

    def z_step(i, _):
        z = jnp.zeros((16,), jnp.int32)
        ctr_v[pl.ds(i * 64, 16)] = z
        ctr_v[pl.ds(i * 64 + 16, 16)] = z
        ctr_v[pl.ds(i * 64 + 32, 16)] = z
        ctr_v[pl.ds(i * 64 + 48, 16)] = z
        return 0

    lax.fori_loop(0, NB // 64, z_step, 0)

    def h_step(i, _):
        k = keys_v[pl.ds(i * 16, 16)]
        plsc.addupdate_scatter(ctr_v, [kbin(k)], ones_i)
        return 0

    lax.fori_loop(0, nv, h_step, 0)

    def s_step(i, carry):
        cnts = [ctr_v[pl.ds(i * 64 + 16 * r, 16)] for r in range(4)]
        css = [plsc.cumsum(c) for c in cnts]
        tots = [jnp.sum(c) for c in cnts]
        for r in range(4):
            excl = css[r] - cnts[r] + carry
            off_v[pl.ds(i * 64 + 16 * r, 16)] = excl
            ctr_v[pl.ds(i * 64 + 16 * r, 16)] = excl
            carry = carry + tots[r]
        return carry

    lax.fori_loop(0, NB // 64, s_step, jnp.int32(0))

    def p_step(i, _):
        k = keys_v[pl.ds(i * 16, 16)]
        g = base + i * 16 + iota
        b = kbin(k)

        def cond(c):
            return jnp.any(c[0])

        def body(c):
            act = c[0]
            plsc.store_scatter(claim_v, [b], iota, mask=act)
            win = (plsc.load_gather(claim_v, [b]) == iota) & act
            pos = plsc.load_gather(ctr_v, [b])
            plsc.store_scatter(sv_v, [pos], k, mask=win)
            plsc.store_scatter(si_v, [pos], g, mask=win)
            plsc.addupdate_scatter(ctr_v, [b], ones_i, mask=win)
            return (act & jnp.logical_not(win),)

        lax.while_loop(cond, body, (jnp.ones((16,), jnp.bool_),))
        return 0

    lax.fori_loop(0, nv, p_step, 0)

    NQ = 4

    def q_step(j, _):
        qs = [xq_v[pl.ds(j * 16 * NQ + 16 * r, 16)] for r in range(NQ)]
        sqs = [(q - vmn) * scale for q in qs]
        p0s = [plsc.load_gather(off_v, [kbin(q)]) for q in qs]
        inf_v = jnp.full((16,), jnp.inf, jnp.float32)
        big_v = jnp.full((16,), 1 << 30, jnp.int32)

        def upd_best(q, act, v, gi, bd, bi):
            d = jnp.abs(q - v)
            u = act & ((d < bd) | ((d == bd) & (gi < bi)))
            return jnp.where(u, d, bd), jnp.where(u, gi, bi)

        def bi_cond(c):
            alive = None
            for (actu, _, actd, _, _, _) in c:
                a = actu | actd
                alive = a if alive is None else (alive | a)
            return jnp.any(alive)

        def one(q, sq, c):
            actu, curu, actd, curd, bd, bi = c
            vu = plsc.load_gather(sv_v, [curu])
            gu = plsc.load_gather(si_v, [curu])
            vd = plsc.load_gather(sv_v, [curd])
            gd = plsc.load_gather(si_v, [curd])
            bd, bi = upd_best(q, actu, vu, gu, bd, bi)
            bd, bi = upd_best(q, actd, vd, gd, bd, bi)
            sbd = bd * scale
            su = (vu - vmn) * scale
            sd = (vd - vmn) * scale
            actu = actu & jnp.logical_not((su - sq) - sbd > 1.05) \
                        & (curu + 1 < nk)
            actd = actd & jnp.logical_not((sq - sd) - sbd > 1.05) \
                        & (curd - 1 >= 0)
            curu = jnp.where(actu, curu + 1, curu)
            curd = jnp.where(actd, curd - 1, curd)
            return actu, curu, actd, curd, bd, bi

        def bi_body(c):
            return tuple(one(qs[r], sqs[r], c[r]) for r in range(NQ))

        def init(p0):
            return (p0 < nk, jnp.minimum(p0, nk - 1),
                    (p0 - 1) >= 0, jnp.maximum(p0 - 1, 0), inf_v, big_v)

        fin = lax.while_loop(bi_cond, bi_body,
                             tuple(init(p0s[r]) for r in range(NQ)))
        for r in range(NQ):
            bd_v[pl.ds(j * 16 * NQ + 16 * r, 16)] = fin[r][4]
            bi_v[pl.ds(j * 16 * NQ + 16 * r, 16)] = fin[r][5]
        return 0

    lax.fori_loop(0, QV // NQ, q_step, 0)

    pltpu.sync_copy(bd_v, outd_hbm.at[wid])
    pltpu.sync_copy(bi_v, outi_hbm.at[wid])


@functools.partial(
    pl.kernel, mesh=_mesh, compiler_params=_params,
    out_type=[jax.ShapeDtypeStruct((NT, Q), jnp.float32),
              jax.ShapeDtypeStruct((NT, Q), jnp.int32)],
    scratch_types=[
        pltpu.VMEM((PER,), jnp.float32),
        pltpu.VMEM((Q,), jnp.float32),
        pltpu.VMEM((PER,), jnp.float32),
        pltpu.VMEM((PER,), jnp.int32),
        pltpu.VMEM((NB,), jnp.int32),
        pltpu.VMEM((NB,), jnp.int32),
        pltpu.VMEM((NB,), jnp.int32),
        pltpu.VMEM((Q,), jnp.float32),
        pltpu.VMEM((Q,), jnp.int32),
    ],
)
def _build_search(x_hbm, keys_hbm, outd_hbm, outi_hbm,
                  keys_v, xq_v, sv_v, si_v, off_v, ctr_v, claim_v, bd_v, bi_v):
    _build_search_body(x_hbm, keys_hbm, outd_hbm, outi_hbm,
                       keys_v, xq_v, sv_v, si_v, off_v, ctr_v, claim_v,
                       bd_v, bi_v)


@functools.partial(
    pl.kernel, mesh=_mesh, compiler_params=_params,
    out_type=jax.ShapeDtypeStruct((Q,), jnp.float32),
    scratch_types=[
        pltpu.VMEM((NT, 32), jnp.float32),
        pltpu.VMEM((NT, 32), jnp.int32),
        pltpu.VMEM((32,), jnp.int32),
        pltpu.VMEM((32,), jnp.float32),
        pltpu.SemaphoreType.DMA,
    ],
)
def _merge_gather(d_hbm, i_hbm, acc_hbm, out_hbm, dbuf, ibuf, win_i, vbuf, sem):
    wid = lax.axis_index("s") * 2 + lax.axis_index("c")
    qbase = wid * 32

    copies = []
    for t in range(NT):
        copies.append(pltpu.async_copy(
            d_hbm.at[t, pl.ds(qbase, 32)], dbuf.at[t], sem))
        copies.append(pltpu.async_copy(
            i_hbm.at[t, pl.ds(qbase, 32)], ibuf.at[t], sem))
    for c in copies:
        c.wait()

    def mix(bd, bi, d, gi):
        u = (d < bd) | ((d == bd) & (gi < bi))
        return jnp.where(u, d, bd), jnp.where(u, gi, bi)

    bd0 = jnp.full((16,), jnp.inf, jnp.float32)
    bd1 = bd0
    bi0 = jnp.full((16,), 1 << 30, jnp.int32)
    bi1 = bi0
    for t in range(NT):
        bd0, bi0 = mix(bd0, bi0, dbuf[t, pl.ds(0, 16)], ibuf[t, pl.ds(0, 16)])
        bd1, bi1 = mix(bd1, bi1, dbuf[t, pl.ds(16, 16)], ibuf[t, pl.ds(16, 16)])
    win_i[pl.ds(0, 16)] = bi0
    win_i[pl.ds(16, 16)] = bi1
    pltpu.async_copy(acc_hbm.at[win_i], vbuf, sem).wait()
    pltpu.sync_copy(vbuf, out_hbm.at[pl.ds(qbase, 32)])


def kernel(x, input_tensor, accuracy_tensor):
    pd, pi = _build_search(x, input_tensor)
    return _merge_gather(pd, pi, accuracy_tensor)
